# speedup vs baseline: 3.2746x; 3.2746x over previous
import functools

import jax
import jax.numpy as jnp
from jax import lax
from jax.experimental import pallas as pl
from jax.experimental.pallas import tpu as pltpu
from jax.experimental.pallas import tpu_sc as plsc

NUM_EMBEDDINGS = 100000
EMBEDDING_DIM = 128

_info = plsc.get_sparse_core_info()
_NC, _NS = _info.num_cores, _info.num_subcores
_NW = _NC * _NS


def _make_gather(B, D, chunk):
    assert B % (_NW * chunk) == 0 and chunk % 8 == 0
    b_per_w = B // _NW
    n_chunks = b_per_w // chunk
    mesh = plsc.VectorSubcoreMesh(core_axis_name="c", subcore_axis_name="s")

    @functools.partial(
        pl.kernel,
        mesh=mesh,
        out_type=jax.ShapeDtypeStruct((B, D), jnp.float32),
        scratch_types=[
            pltpu.VMEM((chunk,), jnp.int32),
            pltpu.VMEM((chunk, D), jnp.float32),
            pltpu.SemaphoreType.DMA,
        ],
    )
    def gather_kernel(ids_hbm, table_hbm, out_hbm, idx_v, rows_v, sem):
        wid = lax.axis_index("s") * _NC + lax.axis_index("c")
        base = wid * b_per_w

        def body(i, carry):
            off = base + i * chunk
            pltpu.sync_copy(ids_hbm.at[pl.ds(off, chunk)], idx_v)
            pltpu.async_copy(table_hbm.at[idx_v], rows_v, sem).wait()
            pltpu.sync_copy(rows_v, out_hbm.at[pl.ds(off, chunk)])
            return carry

        lax.fori_loop(0, n_chunks, body, 0)

    return gather_kernel


_gather = _make_gather(4096 * 50, EMBEDDING_DIM, chunk=640)


def kernel(token_ids, embedding):
    ids_flat = token_ids.reshape(-1).astype(jnp.int32)
    out = _gather(ids_flat, embedding)
    return out.reshape(*token_ids.shape, EMBEDDING_DIM)


# speedup vs baseline: 3.3015x; 1.0082x over previous
import functools

import jax
import jax.numpy as jnp
from jax import lax
from jax.experimental import pallas as pl
from jax.experimental.pallas import tpu as pltpu
from jax.experimental.pallas import tpu_sc as plsc

NUM_EMBEDDINGS = 100000
EMBEDDING_DIM = 128

_info = plsc.get_sparse_core_info()
_NC, _NS = _info.num_cores, _info.num_subcores
_NW = _NC * _NS


def _make_gather(B, D, chunk):
    assert B % (_NW * chunk) == 0 and chunk % 8 == 0
    b_per_w = B // _NW
    n_chunks = b_per_w // chunk
    assert n_chunks % 2 == 0
    n_pairs = n_chunks // 2
    mesh = plsc.VectorSubcoreMesh(core_axis_name="c", subcore_axis_name="s")

    @functools.partial(
        pl.kernel,
        mesh=mesh,
        out_type=jax.ShapeDtypeStruct((B, D), jnp.float32),
        scratch_types=[
            pltpu.VMEM((chunk,), jnp.int32),
            pltpu.VMEM((chunk,), jnp.int32),
            pltpu.VMEM((chunk, D), jnp.float32),
            pltpu.VMEM((chunk, D), jnp.float32),
            pltpu.SemaphoreType.DMA,
            pltpu.SemaphoreType.DMA,
            pltpu.SemaphoreType.DMA,
            pltpu.SemaphoreType.DMA,
        ],
    )
    def gather_kernel(ids_hbm, table_hbm, out_hbm, idx_a, idx_b, rows_a,
                      rows_b, gsem_a, gsem_b, ssem_a, ssem_b):
        wid = lax.axis_index("s") * _NC + lax.axis_index("c")
        base = wid * b_per_w

        def body(j, carry):
            off0 = base + 2 * j * chunk
            off1 = off0 + chunk
            pltpu.sync_copy(ids_hbm.at[pl.ds(off0, chunk)], idx_a)
            cg0 = pltpu.async_copy(table_hbm.at[idx_a], rows_a, gsem_a)
            pltpu.sync_copy(ids_hbm.at[pl.ds(off1, chunk)], idx_b)
            cg1 = pltpu.async_copy(table_hbm.at[idx_b], rows_b, gsem_b)
            cg0.wait()
            cs0 = pltpu.async_copy(rows_a, out_hbm.at[pl.ds(off0, chunk)],
                                   ssem_a)
            cg1.wait()
            cs1 = pltpu.async_copy(rows_b, out_hbm.at[pl.ds(off1, chunk)],
                                   ssem_b)
            cs0.wait()
            cs1.wait()
            return carry

        lax.fori_loop(0, n_pairs, body, 0)

    return gather_kernel


_gather = _make_gather(4096 * 50, EMBEDDING_DIM, chunk=400)


def kernel(token_ids, embedding):
    ids_flat = token_ids.reshape(-1).astype(jnp.int32)
    out = _gather(ids_flat, embedding)
    return out.reshape(*token_ids.shape, EMBEDDING_DIM)


# speedup vs baseline: 5.8015x; 1.7572x over previous
import functools

import jax
import jax.numpy as jnp
from jax import lax
from jax.experimental import pallas as pl
from jax.experimental.pallas import tpu as pltpu
from jax.experimental.pallas import tpu_sc as plsc

NUM_EMBEDDINGS = 100000
EMBEDDING_DIM = 128

_info = plsc.get_sparse_core_info()
_NC, _NS = _info.num_cores, _info.num_subcores
_NW = _NC * _NS


def _make_gather(S, T, D, nseq):
    assert S % (_NW * nseq) == 0 and (nseq * T) % 8 == 0
    s_per_w = S // _NW
    n_chunks = s_per_w // nseq
    assert n_chunks % 2 == 0
    n_pairs = n_chunks // 2
    chunk = nseq * T
    mesh = plsc.VectorSubcoreMesh(core_axis_name="c", subcore_axis_name="s")

    @functools.partial(
        pl.kernel,
        mesh=mesh,
        out_type=jax.ShapeDtypeStruct((S, T, D), jnp.float32),
        scratch_types=[
            pltpu.VMEM((chunk,), jnp.int32),
            pltpu.VMEM((chunk,), jnp.int32),
            pltpu.VMEM((chunk, D), jnp.float32),
            pltpu.VMEM((chunk, D), jnp.float32),
            pltpu.SemaphoreType.DMA,
            pltpu.SemaphoreType.DMA,
            pltpu.SemaphoreType.DMA,
            pltpu.SemaphoreType.DMA,
        ],
    )
    def gather_kernel(ids_hbm, table_hbm, out_hbm, idx_a, idx_b, rows_a,
                      rows_b, gsem_a, gsem_b, ssem_a, ssem_b):
        wid = lax.axis_index("s") * _NC + lax.axis_index("c")
        base = wid * s_per_w * T
        seq_base = wid * s_per_w

        def scatter_chunk(rows, seq0, sem):
            copies = []
            for k in range(nseq):
                copies.append(pltpu.async_copy(
                    rows.at[pl.ds(k * T, T)], out_hbm.at[seq0 + k], sem))
            return copies

        def body(j, carry):
            i0 = 2 * j
            i1 = i0 + 1
            pltpu.sync_copy(ids_hbm.at[pl.ds(base + i0 * chunk, chunk)], idx_a)
            cg0 = pltpu.async_copy(table_hbm.at[idx_a], rows_a, gsem_a)
            pltpu.sync_copy(ids_hbm.at[pl.ds(base + i1 * chunk, chunk)], idx_b)
            cg1 = pltpu.async_copy(table_hbm.at[idx_b], rows_b, gsem_b)
            cg0.wait()
            cs0 = scatter_chunk(rows_a, seq_base + i0 * nseq, ssem_a)
            cg1.wait()
            cs1 = scatter_chunk(rows_b, seq_base + i1 * nseq, ssem_b)
            for c in cs0:
                c.wait()
            for c in cs1:
                c.wait()
            return carry

        lax.fori_loop(0, n_pairs, body, 0)

    return gather_kernel


_gather = _make_gather(4096, 50, EMBEDDING_DIM, nseq=8)


def kernel(token_ids, embedding):
    ids_flat = token_ids.reshape(-1).astype(jnp.int32)
    return _gather(ids_flat, embedding)
